# Initial kernel scaffold; baseline (speedup 1.0000x reference)
import jax
import jax.numpy as jnp
from jax.experimental import pallas as pl


def kernel(x, edge_index, batch, ptr, centrality, W_init, b_init, W0, b0, W1, b1, W2, b2, ln0_g, ln0_b, ln1_g, ln1_b, W_cls, b_cls):
    raise NotImplementedError("write your pallas kernel here")


# speedup vs baseline: 17.7689x; 17.7689x over previous
import functools

import jax
import jax.numpy as jnp
from jax import lax
from jax.experimental import pallas as pl
from jax.experimental.pallas import tpu as pltpu
from jax.experimental.pallas import tpu_sc as plsc

N = 10000
N_PAD = 10240
E = 320000
H = 128
G = 100
C = 10

NC = 2
NS = 16
NW = NC * NS
EPT = E // NW
CH = 125
NK = EPT // CH
RPS = N_PAD // NS

_mesh = plsc.VectorSubcoreMesh(core_axis_name="c", subcore_axis_name="s")


@functools.partial(
    pl.kernel,
    out_type=jax.ShapeDtypeStruct((NC, N_PAD, H), jnp.float32),
    mesh=_mesh,
    scratch_types=[
        pltpu.VMEM((NK, CH), jnp.int32),
        pltpu.VMEM((CH, H), jnp.float32),
        pltpu.VMEM_SHARED((N_PAD, H), jnp.float32),
    ],
)
def _sc_degree(col_hbm, z_hbm, ones_hbm, deg_hbm, colb, onesb, degs):
    c = lax.axis_index("c")
    s = lax.axis_index("s")
    w = c * NS + s
    pltpu.sync_copy(z_hbm, degs.at[pl.ds(s * RPS, RPS)])
    pltpu.sync_copy(col_hbm.at[pl.ds(w * NK, NK)], colb)
    pltpu.sync_copy(ones_hbm, onesb)
    plsc.subcore_barrier()

    def body(k, carry):
        pltpu.sync_copy(onesb, degs.at[colb.at[k]], add=True)
        return carry

    lax.fori_loop(0, NK, body, 0)
    plsc.subcore_barrier()
    pltpu.sync_copy(degs.at[pl.ds(s * RPS, RPS)],
                    deg_hbm.at[c, pl.ds(s * RPS, RPS)])


@functools.partial(
    pl.kernel,
    out_type=jax.ShapeDtypeStruct((NC, N_PAD, H), jnp.float32),
    mesh=_mesh,
    scratch_types=[
        pltpu.VMEM((NK, CH), jnp.int32),
        pltpu.VMEM((NK, CH), jnp.int32),
        pltpu.VMEM((CH, H), jnp.float32),
        pltpu.VMEM_SHARED((N_PAD, H), jnp.float32),
        pltpu.SemaphoreType.DMA,
    ],
)
def _sc_scatter(y_hbm, row_hbm, col_hbm, z_hbm, acc_hbm,
                rowb, colb, rows, accs, sem):
    c = lax.axis_index("c")
    s = lax.axis_index("s")
    w = c * NS + s
    pltpu.sync_copy(z_hbm, accs.at[pl.ds(s * RPS, RPS)])
    pltpu.sync_copy(row_hbm.at[pl.ds(w * NK, NK)], rowb)
    pltpu.sync_copy(col_hbm.at[pl.ds(w * NK, NK)], colb)
    plsc.subcore_barrier()

    def body(k, carry):
        pltpu.async_copy(y_hbm.at[rowb.at[k]], rows, sem).wait()
        pltpu.sync_copy(rows, accs.at[colb.at[k]], add=True)
        return carry

    lax.fori_loop(0, NK, body, 0)
    plsc.subcore_barrier()
    pltpu.sync_copy(accs.at[pl.ds(s * RPS, RPS)],
                    acc_hbm.at[c, pl.ds(s * RPS, RPS)])


def _dinv(dp_ref):
    return lax.rsqrt(1.0 + dp_ref[0, :, 0:1] + dp_ref[1, :, 0:1])


def _m0_body(x_ref, dp_ref, wi_ref, bi_ref, w0_ref, y_ref):
    h = jnp.dot(x_ref[...], wi_ref[...],
                preferred_element_type=jnp.float32) + bi_ref[...]
    y_ref[...] = _dinv(dp_ref) * jnp.dot(
        h, w0_ref[...], preferred_element_type=jnp.float32)


def _m12_body(acc_ref, y_ref, dp_ref, b_ref, g_ref, bb_ref, w_ref, out_ref):
    dinv = _dinv(dp_ref)
    h = dinv * (acc_ref[0] + acc_ref[1] + y_ref[...]) + b_ref[...]
    h = jnp.maximum(h, 0.0)
    m = jnp.mean(h, axis=1, keepdims=True)
    v = jnp.mean((h - m) ** 2, axis=1, keepdims=True)
    h = (h - m) * lax.rsqrt(v + 1e-5) * g_ref[...] + bb_ref[...]
    out_ref[...] = dinv * jnp.dot(
        h, w_ref[...], preferred_element_type=jnp.float32)


BN = 2000
BN3 = 1000


def _m3_body(acc_ref, y_ref, dp_ref, b_ref, batch_ref, wc_ref, bc_ref,
             emb_ref, logp_ref, sums, cnts):
    i = pl.program_id(0)
    dinv = _dinv(dp_ref)
    h = dinv * (acc_ref[0] + acc_ref[1] + y_ref[...]) + b_ref[...]
    onehot = (jax.lax.broadcasted_iota(jnp.int32, (BN3, G), 1)
              == batch_ref[...]).astype(jnp.float32)
    ps = jax.lax.dot_general(onehot, h, (((0,), (0,)), ((), ())),
                             preferred_element_type=jnp.float32)
    pc = jax.lax.dot_general(onehot, jnp.ones_like(h),
                             (((0,), (0,)), ((), ())),
                             preferred_element_type=jnp.float32)

    @pl.when(i == 0)
    def _():
        sums[...] = ps
        cnts[...] = pc

    @pl.when(i > 0)
    def _():
        sums[...] += ps
        cnts[...] += pc

    @pl.when(i == pl.num_programs(0) - 1)
    def _():
        mean = sums[...] / jnp.maximum(cnts[...], 1.0)
        emb = jnp.dot(mean, wc_ref[...],
                      preferred_element_type=jnp.float32) + bc_ref[...]
        mx = jnp.max(emb, axis=1, keepdims=True)
        ls = jnp.log(jnp.sum(jnp.exp(emb - mx), axis=1, keepdims=True))
        emb_ref[...] = emb
        logp_ref[...] = emb - mx - ls


def _full(shape):
    return pl.BlockSpec(shape, lambda i: (0,) * len(shape))


def _m0(x, dp, wi, bi, w0):
    return pl.pallas_call(
        _m0_body,
        grid=(N // BN,),
        in_specs=[
            pl.BlockSpec((BN, H), lambda i: (i, 0)),
            pl.BlockSpec((NC, BN, H), lambda i: (0, i, 0)),
            _full((H, H)),
            _full((1, H)),
            _full((H, H)),
        ],
        out_specs=pl.BlockSpec((BN, H), lambda i: (i, 0)),
        out_shape=jax.ShapeDtypeStruct((N, H), jnp.float32),
    )(x, dp, wi, bi, w0)


def _m12(acc, y, dp, b, g, bb, w):
    return pl.pallas_call(
        _m12_body,
        grid=(N // BN,),
        in_specs=[
            pl.BlockSpec((NC, BN, H), lambda i: (0, i, 0)),
            pl.BlockSpec((BN, H), lambda i: (i, 0)),
            pl.BlockSpec((NC, BN, H), lambda i: (0, i, 0)),
            _full((1, H)),
            _full((1, H)),
            _full((1, H)),
            _full((H, H)),
        ],
        out_specs=pl.BlockSpec((BN, H), lambda i: (i, 0)),
        out_shape=jax.ShapeDtypeStruct((N, H), jnp.float32),
    )(acc, y, dp, b, g, bb, w)


def _m3(acc, y, dp, b, batch2d, wc, bc):
    return pl.pallas_call(
        _m3_body,
        grid=(N // BN3,),
        in_specs=[
            pl.BlockSpec((NC, BN3, H), lambda i: (0, i, 0)),
            pl.BlockSpec((BN3, H), lambda i: (i, 0)),
            pl.BlockSpec((NC, BN3, H), lambda i: (0, i, 0)),
            _full((1, H)),
            pl.BlockSpec((BN3, 1), lambda i: (i, 0)),
            _full((H, C)),
            _full((1, C)),
        ],
        out_specs=[_full((G, C)), _full((G, C))],
        out_shape=[
            jax.ShapeDtypeStruct((G, C), jnp.float32),
            jax.ShapeDtypeStruct((G, C), jnp.float32),
        ],
        scratch_shapes=[
            pltpu.VMEM((G, H), jnp.float32),
            pltpu.VMEM((G, H), jnp.float32),
        ],
    )(acc, y, dp, b, batch2d, wc, bc)


def kernel(x, edge_index, batch, ptr, centrality, W_init, b_init,
           W0, b0, W1, b1, W2, b2, ln0_g, ln0_b, ln1_g, ln1_b,
           W_cls, b_cls):
    row = edge_index[0].astype(jnp.int32).reshape(NW * NK, CH)
    col = edge_index[1].astype(jnp.int32).reshape(NW * NK, CH)
    zh = jnp.zeros((RPS, H), jnp.float32)
    bi = b_init.reshape(1, H)
    b0r, b1r, b2r = b0.reshape(1, H), b1.reshape(1, H), b2.reshape(1, H)
    g0, bb0 = ln0_g.reshape(1, H), ln0_b.reshape(1, H)
    g1, bb1 = ln1_g.reshape(1, H), ln1_b.reshape(1, H)
    bc = b_cls.reshape(1, C)
    batch2d = batch.astype(jnp.int32).reshape(N, 1)

    dp = _sc_degree(col, zh, jnp.ones((CH, H), jnp.float32))
    y0 = _m0(x, dp, W_init, bi, W0)
    acc = _sc_scatter(y0, row, col, zh)
    y1 = _m12(acc, y0, dp, b0r, g0, bb0, W1)
    acc = _sc_scatter(y1, row, col, zh)
    y2 = _m12(acc, y1, dp, b1r, g1, bb1, W2)
    acc = _sc_scatter(y2, row, col, zh)
    emb, logp = _m3(acc, y2, dp, b2r, batch2d, W_cls, bc)
    return (emb, logp)
